# baseline (speedup 1.0000x reference)
import functools

import jax
import jax.numpy as jnp
from jax import lax
from jax.experimental import pallas as pl
from jax.experimental.pallas import tpu as pltpu
from jax.experimental.pallas import tpu_sc as plsc

NC = 2
NS = 16
L = 16
NW = NC * NS
LANES = 128
HALF = 512
RING = 5
SEG = 128


def _sweep_kernel(n_tokens, hidden, vocab):
    n_per_w = n_tokens // NW
    n_half = n_per_w // HALF
    hq = hidden // L
    max_col = ((vocab + LANES - 1) // LANES) * LANES - SEG
    mesh = plsc.VectorSubcoreMesh(
        core_axis_name="c", subcore_axis_name="s",
        num_cores=NC, num_subcores=NS,
    )

    @functools.partial(
        pl.kernel,
        out_type=jax.ShapeDtypeStruct((n_tokens * hidden,), jnp.float32),
        mesh=mesh,
        compiler_params=pltpu.CompilerParams(use_tc_tiling_on_sc=True,
                                             needs_layout_passes=False),
        scratch_types=[
            pltpu.VMEM((HALF + L,), jnp.int32),
            pltpu.VMEM((HALF + 2 * L,), jnp.int32),
            pltpu.VMEM_SHARED((NS, 2, hidden, SEG), jnp.float32),
            pltpu.VMEM((2, hidden, SEG), jnp.float32),
            pltpu.VMEM((HALF * hidden,), jnp.float32),
            pltpu.SemaphoreType.DMA((2,)),
            pltpu.SemaphoreType.DMA((2,)),
            pltpu.SemaphoreType.DMA,
        ],
    )
    def body(ids_hbm, table_hbm, out_hbm, ids_s, runs_v, ring_sh, ring_v, r_v,
             gsem, hsem, osem):
        wid = lax.axis_index("s") * NC + lax.axis_index("c")
        base = wid * n_per_w
        lane = lax.iota(jnp.int32, L)
        m0 = lane == 0

        def sread(ref, i):
            return ref[pl.ds(i, L)][0]

        def swrite(ref, i, val):
            plsc.store_scatter(ref, [jnp.broadcast_to(i, (L,))],
                               jnp.broadcast_to(val, (L,)), mask=m0)

        def seg_col(j):
            return jnp.minimum(j * SEG, max_col)

        def slab_src(j):
            col = pl.multiple_of(seg_col(j), LANES)
            return table_hbm.at[:, pl.ds(col, SEG)]

        def out_dst(half):
            return out_hbm.at[pl.ds((base + half * HALF) * hidden,
                                    HALF * hidden)]

        for half in range(n_half):
            hbase = base + half * HALF
            pltpu.sync_copy(ids_hbm.at[pl.ds(hbase, HALF)],
                            ids_s.at[pl.ds(0, HALF)])
            ids_s[pl.ds(HALF, L)] = jnp.full((L,), 2 * vocab, jnp.int32)

            def rle(k, carry):
                nruns, run_start, jk = carry
                jk1 = sread(ids_s, k + 1) // SEG
                swrite(runs_v, nruns, jk * 2048 + run_start)
                isend = (jk1 != jk).astype(jnp.int32)
                return (nruns + isend,
                        isend * (k + 1) + (1 - isend) * run_start,
                        jk1)

            j_first = sread(ids_s, 0) // SEG
            nruns, _, _ = lax.fori_loop(
                0, HALF, rle, (jnp.int32(0), jnp.int32(0), j_first))
            for i in range(RING):
                swrite(runs_v, nruns + i, jnp.int32(HALF))

            sid = lax.axis_index("s")

            def hop1(d):
                jd = sread(runs_v, d) // 2048
                pltpu.async_copy(slab_src(jd), ring_sh.at[sid, lax.rem(d, 2)],
                                 gsem.at[lax.rem(d, 2)])

            def hop1_wait(d):
                s2 = lax.rem(d, 2)
                pltpu.make_async_copy(slab_src(jnp.int32(0)),
                                      ring_sh.at[sid, s2],
                                      gsem.at[s2]).wait()

            def hop2(d):
                p = lax.rem(d, 2)
                pltpu.async_copy(ring_sh.at[sid, p], ring_v.at[p],
                                 hsem.at[p])

            def hop2_wait(d):
                p = lax.rem(d, 2)
                pltpu.make_async_copy(ring_sh.at[sid, 0], ring_v.at[p],
                                      hsem.at[p]).wait()

            hop1(jnp.int32(0))
            hop1(jnp.int32(1))
            hop1_wait(jnp.int32(0))
            hop2(jnp.int32(0))

            def run_iter(d, carry):
                rv = runs_v[pl.ds(d, L)]
                rd = rv[0]
                j = rd // 2048
                k1 = rd - j * 2048
                rn = rv[1]
                k2 = rn - (rn // 2048) * 2048
                p = lax.rem(d, 2)
                hop2_wait(d)
                hop1(d + 2)
                hop1_wait(d + 1)
                hop2(d + 1)
                col = seg_col(j)

                def extract(kk, carry2):
                    c = sread(ids_s, kk) - col
                    cb = jnp.broadcast_to(c, (L,))
                    for q in range(hq):
                        vals = plsc.load_gather(
                            ring_v.at[p], [lane + q * L, cb])
                        r_v[pl.ds(kk * hidden + q * L, L)] = vals
                    return carry2

                lax.fori_loop(k1, k2, extract, 0)
                return carry

            lax.fori_loop(0, nruns, run_iter, jnp.int32(0))
            hop2_wait(nruns)
            hop1_wait(nruns + 1)

            pltpu.async_copy(r_v, out_dst(half), osem)
            if half + 1 < n_half:
                pltpu.make_async_copy(r_v, out_dst(half), osem).wait()
        pltpu.make_async_copy(r_v, out_dst(n_half - 1), osem).wait()

    return body


def _unpermute_kernel(n_tokens, seq, hidden, chunk):
    n_per_w = n_tokens // NW
    n_chunks = n_per_w // chunk
    mesh = plsc.VectorSubcoreMesh(
        core_axis_name="c", subcore_axis_name="s",
        num_cores=NC, num_subcores=NS,
    )

    @functools.partial(
        pl.kernel,
        out_type=jax.ShapeDtypeStruct((n_tokens, hidden), jnp.float32),
        mesh=mesh,
        compiler_params=pltpu.CompilerParams(use_tc_tiling_on_sc=False),
        scratch_types=[
            pltpu.VMEM((n_chunks, chunk), jnp.int32),
            pltpu.VMEM((n_chunks, chunk), jnp.int32),
            pltpu.VMEM((chunk, hidden), jnp.float32),
            pltpu.VMEM((chunk, hidden), jnp.float32),
            pltpu.SemaphoreType.DMA,
            pltpu.SemaphoreType.DMA,
        ],
    )
    def body(res_hbm, order_hbm, pos_hbm, out_hbm, o_v, s_v, rows_v, pos_v,
             psem, osem):
        wid = lax.axis_index("s") * NC + lax.axis_index("c")
        base = wid * n_per_w
        for c in range(n_chunks):
            off = c * chunk
            pltpu.sync_copy(order_hbm.at[pl.ds(base + off, chunk)], o_v.at[c])

            def mod_seq(i, carry):
                sl = pl.ds(i * L, L)
                s_v[c, sl] = lax.rem(o_v[c, sl], seq)
                return carry
            lax.fori_loop(0, chunk // L, mod_seq, 0)

            pltpu.sync_copy(res_hbm.at[pl.ds(base + off, chunk)], rows_v)
            pltpu.async_copy(pos_hbm.at[s_v.at[c]], pos_v, psem).wait()

            def add_row(r, carry):
                for q in range(hidden // L):
                    sl = pl.ds(q * L, L)
                    rows_v[r, sl] = rows_v[r, sl] + pos_v[r, sl]
                return carry
            lax.fori_loop(0, chunk, add_row, 0)

            pltpu.async_copy(rows_v, out_hbm.at[o_v.at[c]], osem)
            pltpu.make_async_copy(rows_v, out_hbm.at[o_v.at[c]], osem).wait()

    return body


def kernel(token_ids, table, pos_emb):
    b, s = token_ids.shape
    v, h = table.shape
    n = b * s
    ids_flat = token_ids.reshape(n).astype(jnp.int32)
    iota = lax.iota(jnp.int32, n)
    sorted_ids, order = lax.sort((ids_flat, iota), num_keys=1)
    res_flat = _sweep_kernel(n, h, v)(sorted_ids, table.T)
    out = _unpermute_kernel(n, s, h, chunk=512)(
        res_flat.reshape(n, h), order, pos_emb)
    return out.reshape(b, s, h)


# speedup vs baseline: 1.3467x; 1.3467x over previous
import functools

import jax
import jax.numpy as jnp
from jax import lax
from jax.experimental import pallas as pl
from jax.experimental.pallas import tpu as pltpu
from jax.experimental.pallas import tpu_sc as plsc

NC = 2
NS = 16
L = 16
NW = NC * NS
LANES = 128
HALF = 512
RING = 5
SEG = 256


def _sweep_kernel(n_tokens, hidden, vocab):
    n_per_w = n_tokens // NW
    n_half = n_per_w // HALF
    hq = hidden // L
    max_col = ((vocab + LANES - 1) // LANES) * LANES - SEG
    mesh = plsc.VectorSubcoreMesh(
        core_axis_name="c", subcore_axis_name="s",
        num_cores=NC, num_subcores=NS,
    )

    @functools.partial(
        pl.kernel,
        out_type=jax.ShapeDtypeStruct((n_tokens * hidden,), jnp.float32),
        mesh=mesh,
        compiler_params=pltpu.CompilerParams(use_tc_tiling_on_sc=True,
                                             needs_layout_passes=False),
        scratch_types=[
            pltpu.VMEM((HALF + L,), jnp.int32),
            pltpu.VMEM((HALF + 2 * L,), jnp.int32),
            pltpu.VMEM((RING, hidden, SEG), jnp.float32),
            pltpu.VMEM((HALF * hidden,), jnp.float32),
            pltpu.SemaphoreType.DMA((RING,)),
            pltpu.SemaphoreType.DMA,
        ],
    )
    def body(ids_hbm, table_hbm, out_hbm, ids_s, runs_v, ring_v, r_v, gsem,
             osem):
        wid = lax.axis_index("s") * NC + lax.axis_index("c")
        base = wid * n_per_w
        lane = lax.iota(jnp.int32, L)
        m0 = lane == 0

        def sread(ref, i):
            return ref[pl.ds(i, L)][0]

        def swrite(ref, i, val):
            plsc.store_scatter(ref, [jnp.broadcast_to(i, (L,))],
                               jnp.broadcast_to(val, (L,)), mask=m0)

        def seg_col(j):
            return jnp.minimum(j * SEG, max_col)

        def slab_src(j):
            col = pl.multiple_of(seg_col(j), LANES)
            return table_hbm.at[:, pl.ds(col, SEG)]

        def out_dst(half):
            return out_hbm.at[pl.ds((base + half * HALF) * hidden,
                                    HALF * hidden)]

        for half in range(n_half):
            hbase = base + half * HALF
            pltpu.sync_copy(ids_hbm.at[pl.ds(hbase, HALF)],
                            ids_s.at[pl.ds(0, HALF)])
            ids_s[pl.ds(HALF, L)] = jnp.full((L,), 2 * vocab, jnp.int32)

            def rle(k, carry):
                nruns, run_start, jk = carry
                jk1 = sread(ids_s, k + 1) // SEG
                swrite(runs_v, nruns, jk * 2048 + run_start)
                isend = (jk1 != jk).astype(jnp.int32)
                return (nruns + isend,
                        isend * (k + 1) + (1 - isend) * run_start,
                        jk1)

            j_first = sread(ids_s, 0) // SEG
            nruns, _, _ = lax.fori_loop(
                0, HALF, rle, (jnp.int32(0), jnp.int32(0), j_first))
            for i in range(RING):
                swrite(runs_v, nruns + i, jnp.int32(HALF))
            if half > 0:
                pltpu.make_async_copy(r_v, out_dst(half - 1), osem).wait()

            def prime(d, carry):
                jd = sread(runs_v, d) // 2048
                pltpu.async_copy(slab_src(jd), ring_v.at[d], gsem.at[d])
                return carry
            lax.fori_loop(0, RING - 1, prime, 0)

            def run_iter(d, carry):
                slot = lax.rem(d, RING)
                rv = runs_v[pl.ds(d, L)]
                rd = rv[0]
                rn = rv[1]
                j = rd // 2048
                k1 = rd - j * 2048
                k2 = rn - (rn // 2048) * 2048
                fslot = lax.rem(d + RING - 1, RING)
                jf = sread(runs_v, d + RING - 1) // 2048
                pltpu.async_copy(slab_src(jf), ring_v.at[fslot],
                                 gsem.at[fslot])
                pltpu.make_async_copy(slab_src(j), ring_v.at[slot],
                                      gsem.at[slot]).wait()

                col = seg_col(j)

                def extract(kk, carry2):
                    c = sread(ids_s, kk) - col
                    cb = jnp.broadcast_to(c, (L,))
                    for q in range(hq):
                        vals = plsc.load_gather(
                            ring_v.at[slot], [lane + q * L, cb])
                        r_v[pl.ds(kk * hidden + q * L, L)] = vals
                    return carry2

                lax.fori_loop(k1, k2, extract, 0)
                return carry

            lax.fori_loop(0, nruns, run_iter, jnp.int32(0))

            def post_drain(i, carry):
                sl = lax.rem(nruns + i, RING)
                pltpu.make_async_copy(slab_src(jnp.int32(0)),
                                      ring_v.at[sl], gsem.at[sl]).wait()
                return carry
            lax.fori_loop(0, RING - 1, post_drain, 0)

            pltpu.async_copy(r_v, out_dst(half), osem)
        pltpu.make_async_copy(r_v, out_dst(n_half - 1), osem).wait()

    return body


def _unpermute_kernel(n_tokens, seq, hidden, chunk):
    n_per_w = n_tokens // NW
    n_chunks = n_per_w // chunk
    mesh = plsc.VectorSubcoreMesh(
        core_axis_name="c", subcore_axis_name="s",
        num_cores=NC, num_subcores=NS,
    )

    @functools.partial(
        pl.kernel,
        out_type=jax.ShapeDtypeStruct((n_tokens, hidden), jnp.float32),
        mesh=mesh,
        compiler_params=pltpu.CompilerParams(use_tc_tiling_on_sc=False),
        scratch_types=[
            pltpu.VMEM((n_chunks, chunk), jnp.int32),
            pltpu.VMEM((n_chunks, chunk), jnp.int32),
            pltpu.VMEM((chunk, hidden), jnp.float32),
            pltpu.VMEM((chunk, hidden), jnp.float32),
            pltpu.SemaphoreType.DMA,
            pltpu.SemaphoreType.DMA,
        ],
    )
    def body(res_hbm, order_hbm, pos_hbm, out_hbm, o_v, s_v, rows_v, pos_v,
             psem, osem):
        wid = lax.axis_index("s") * NC + lax.axis_index("c")
        base = wid * n_per_w
        for c in range(n_chunks):
            off = c * chunk
            pltpu.sync_copy(order_hbm.at[pl.ds(base + off, chunk)], o_v.at[c])

            def mod_seq(i, carry):
                sl = pl.ds(i * L, L)
                s_v[c, sl] = lax.rem(o_v[c, sl], seq)
                return carry
            lax.fori_loop(0, chunk // L, mod_seq, 0)

            pltpu.sync_copy(res_hbm.at[pl.ds(base + off, chunk)], rows_v)
            pltpu.async_copy(pos_hbm.at[s_v.at[c]], pos_v, psem).wait()

            def add_row(r, carry):
                for q in range(hidden // L):
                    sl = pl.ds(q * L, L)
                    rows_v[r, sl] = rows_v[r, sl] + pos_v[r, sl]
                return carry
            lax.fori_loop(0, chunk, add_row, 0)

            pltpu.async_copy(rows_v, out_hbm.at[o_v.at[c]], osem)
            pltpu.make_async_copy(rows_v, out_hbm.at[o_v.at[c]], osem).wait()

    return body


def kernel(token_ids, table, pos_emb):
    b, s = token_ids.shape
    v, h = table.shape
    n = b * s
    ids_flat = token_ids.reshape(n).astype(jnp.int32)
    iota = lax.iota(jnp.int32, n)
    sorted_ids, order = lax.sort((ids_flat, iota), num_keys=1)
    res_flat = _sweep_kernel(n, h, v)(sorted_ids, table.T)
    out = _unpermute_kernel(n, s, h, chunk=512)(
        res_flat.reshape(n, h), order, pos_emb)
    return out.reshape(b, s, h)


# speedup vs baseline: 1.3773x; 1.0228x over previous
import functools

import jax
import jax.numpy as jnp
from jax import lax
from jax.experimental import pallas as pl
from jax.experimental.pallas import tpu as pltpu
from jax.experimental.pallas import tpu_sc as plsc

NC = 2
NS = 16
L = 16
NW = NC * NS
LANES = 128
HALF = 512
RING = 5
SEG = 256


def _sweep_kernel(n_tokens, hidden, vocab):
    n_per_w = n_tokens // NW
    n_half = n_per_w // HALF
    hq = hidden // L
    max_col = ((vocab + LANES - 1) // LANES) * LANES - SEG
    mesh = plsc.VectorSubcoreMesh(
        core_axis_name="c", subcore_axis_name="s",
        num_cores=NC, num_subcores=NS,
    )

    @functools.partial(
        pl.kernel,
        out_type=jax.ShapeDtypeStruct((n_tokens * hidden,), jnp.float32),
        mesh=mesh,
        compiler_params=pltpu.CompilerParams(use_tc_tiling_on_sc=True,
                                             needs_layout_passes=False),
        scratch_types=[
            pltpu.VMEM((HALF + L,), jnp.int32),
            pltpu.VMEM((HALF + 2 * L,), jnp.int32),
            pltpu.VMEM((RING, hidden, SEG), jnp.float32),
            pltpu.VMEM((HALF * hidden,), jnp.float32),
            pltpu.SemaphoreType.DMA((RING,)),
            pltpu.SemaphoreType.DMA,
        ],
    )
    def body(ids_hbm, table_hbm, out_hbm, ids_s, runs_v, ring_v, r_v, gsem,
             osem):
        wid = lax.axis_index("s") * NC + lax.axis_index("c")
        base = wid * n_per_w
        lane = lax.iota(jnp.int32, L)
        m0 = lane == 0

        def sread(ref, i):
            return ref[pl.ds(i, L)][0]

        def swrite(ref, i, val):
            plsc.store_scatter(ref, [jnp.broadcast_to(i, (L,))],
                               jnp.broadcast_to(val, (L,)), mask=m0)

        def seg_col(j):
            return jnp.minimum(j * SEG, max_col)

        def slab_src(j):
            col = pl.multiple_of(seg_col(j), LANES)
            return table_hbm.at[:, pl.ds(col, SEG)]

        def out_dst(half):
            return out_hbm.at[pl.ds((base + half * HALF) * hidden,
                                    HALF * hidden)]

        for half in range(n_half):
            hbase = base + half * HALF
            pltpu.sync_copy(ids_hbm.at[pl.ds(hbase, HALF)],
                            ids_s.at[pl.ds(0, HALF)])
            ids_s[pl.ds(HALF, L)] = jnp.full((L,), 2 * vocab, jnp.int32)

            def rle(k, carry):
                nruns, run_start, jk = carry
                jk1 = sread(ids_s, k + 1) // SEG
                swrite(runs_v, nruns, jk * 2048 + run_start)
                isend = (jk1 != jk).astype(jnp.int32)
                return (nruns + isend,
                        isend * (k + 1) + (1 - isend) * run_start,
                        jk1)

            j_first = sread(ids_s, 0) // SEG
            nruns, _, _ = lax.fori_loop(
                0, HALF, rle, (jnp.int32(0), jnp.int32(0), j_first))
            for i in range(RING):
                swrite(runs_v, nruns + i, jnp.int32(HALF))
            if half > 0:
                pltpu.make_async_copy(r_v, out_dst(half - 1), osem).wait()

            def prime(d, carry):
                jd = sread(runs_v, d) // 2048
                pltpu.async_copy(slab_src(jd), ring_v.at[d], gsem.at[d])
                return carry
            lax.fori_loop(0, RING - 1, prime, 0)

            def run_iter(d, carry):
                slot = lax.rem(d, RING)
                rv = runs_v[pl.ds(d, L)]
                rd = rv[0]
                rn = rv[1]
                j = rd // 2048
                k1 = rd - j * 2048
                k2 = rn - (rn // 2048) * 2048
                fslot = lax.rem(d + RING - 1, RING)
                jf = sread(runs_v, d + RING - 1) // 2048
                pltpu.async_copy(slab_src(jf), ring_v.at[fslot],
                                 gsem.at[fslot])
                pltpu.make_async_copy(slab_src(j), ring_v.at[slot],
                                      gsem.at[slot]).wait()

                col = seg_col(j)

                def extract(kk, carry2):
                    c = sread(ids_s, kk) - col
                    cb = jnp.broadcast_to(c, (L,))
                    for q in range(hq):
                        vals = plsc.load_gather(
                            ring_v.at[slot], [lane + q * L, cb])
                        r_v[pl.ds(kk * hidden + q * L, L)] = vals
                    return carry2

                lax.fori_loop(k1, k2, extract, 0)
                return carry

            lax.fori_loop(0, nruns, run_iter, jnp.int32(0))

            def post_drain(i, carry):
                sl = lax.rem(nruns + i, RING)
                pltpu.make_async_copy(slab_src(jnp.int32(0)),
                                      ring_v.at[sl], gsem.at[sl]).wait()
                return carry
            lax.fori_loop(0, RING - 1, post_drain, 0)

            pltpu.async_copy(r_v, out_dst(half), osem)
        pltpu.make_async_copy(r_v, out_dst(n_half - 1), osem).wait()

    return body


def _unpermute_kernel(n_tokens, seq, hidden, chunk):
    n_per_w = n_tokens // NW
    n_chunks = n_per_w // chunk
    mesh = plsc.VectorSubcoreMesh(
        core_axis_name="c", subcore_axis_name="s",
        num_cores=NC, num_subcores=NS,
    )

    @functools.partial(
        pl.kernel,
        out_type=jax.ShapeDtypeStruct((n_tokens, hidden), jnp.float32),
        mesh=mesh,
        compiler_params=pltpu.CompilerParams(use_tc_tiling_on_sc=False),
        scratch_types=[
            pltpu.VMEM((n_chunks, chunk), jnp.int32),
            pltpu.VMEM((n_chunks, chunk), jnp.int32),
            pltpu.VMEM((2, chunk, hidden), jnp.float32),
            pltpu.VMEM((2, chunk, hidden), jnp.float32),
            pltpu.SemaphoreType.DMA((2,)),
            pltpu.SemaphoreType.DMA((2,)),
            pltpu.SemaphoreType.DMA((2,)),
        ],
    )
    def body(res_hbm, order_hbm, pos_hbm, out_hbm, o_v, s_v, rows_v, pos_v,
             rsem, psem, osem):
        wid = lax.axis_index("s") * NC + lax.axis_index("c")
        base = wid * n_per_w

        def res_src(c):
            return res_hbm.at[pl.ds(base + c * chunk, chunk)]

        def prep_a(c):
            pltpu.sync_copy(order_hbm.at[pl.ds(base + c * chunk, chunk)],
                            o_v.at[c])

            def mod_seq(i, carry):
                sl = pl.ds(i * L, L)
                s_v[c, sl] = lax.rem(o_v[c, sl], seq)
                return carry
            lax.fori_loop(0, chunk // L, mod_seq, 0)

        def prep_b(c):
            p = c % 2
            if c >= 2:
                pltpu.make_async_copy(rows_v.at[p],
                                      out_hbm.at[o_v.at[c - 2]],
                                      osem.at[p]).wait()
            pltpu.async_copy(res_src(c), rows_v.at[p], rsem.at[p])
            pltpu.async_copy(pos_hbm.at[s_v.at[c]], pos_v.at[p], psem.at[p])

        def consume(c):
            p = c % 2
            pltpu.make_async_copy(res_src(c), rows_v.at[p],
                                  rsem.at[p]).wait()
            pltpu.make_async_copy(pos_hbm.at[s_v.at[c]], pos_v.at[p],
                                  psem.at[p]).wait()

            def add_row(r, carry):
                for q in range(hidden // L):
                    sl = pl.ds(q * L, L)
                    rows_v[p, r, sl] = rows_v[p, r, sl] + pos_v[p, r, sl]
                return carry
            lax.fori_loop(0, chunk, add_row, 0)
            pltpu.async_copy(rows_v.at[p], out_hbm.at[o_v.at[c]], osem.at[p])

        prep_a(0)
        prep_b(0)
        for c in range(1, n_chunks):
            prep_a(c)
            prep_b(c)
            consume(c - 1)
        consume(n_chunks - 1)
        for c in range(n_chunks - 2, n_chunks):
            pltpu.make_async_copy(rows_v.at[c % 2], out_hbm.at[o_v.at[c]],
                                  osem.at[c % 2]).wait()

    return body


def kernel(token_ids, table, pos_emb):
    b, s = token_ids.shape
    v, h = table.shape
    n = b * s
    ids_flat = token_ids.reshape(n).astype(jnp.int32)
    iota = lax.iota(jnp.int32, n)
    sorted_ids, order = lax.sort((ids_flat, iota), num_keys=1)
    res_flat = _sweep_kernel(n, h, v)(sorted_ids, table.T)
    out = _unpermute_kernel(n, s, h, chunk=256)(
        res_flat.reshape(n, h), order, pos_emb)
    return out.reshape(b, s, h)
